# Initial kernel scaffold; baseline (speedup 1.0000x reference)
import functools

import jax
import jax.numpy as jnp
from jax import lax
from jax.experimental import pallas as pl
from jax.experimental.pallas import tpu as pltpu
from jax.experimental.pallas import tpu_sc as plsc

CODEBOOK_SIZE = 8192
CODE_DIM = 32
N_TOKENS = 16384

TM = 256


def _argmin_body(x_ref, cb_ref, idx_ref):
    x = x_ref[...]
    cb = cb_ref[...]
    ab = lax.dot_general(
        x, cb, (((1,), (1,)), ((), ())),
        preferred_element_type=jnp.float32,
    )
    a2 = jnp.sum(x * x, axis=1, keepdims=True)
    b2 = jnp.sum(cb * cb, axis=1)[None, :]
    dist = (a2 - 2.0 * ab) + b2
    m = jnp.min(dist, axis=1, keepdims=True)
    iota = lax.broadcasted_iota(jnp.int32, dist.shape, 1)
    bidx = jnp.min(jnp.where(dist == m, iota, CODEBOOK_SIZE), axis=1)
    idx_ref[...] = bidx.astype(jnp.int32)


def _argmin_indices(x, codebook):
    grid = (N_TOKENS // TM,)
    return pl.pallas_call(
        _argmin_body,
        grid=grid,
        in_specs=[
            pl.BlockSpec((TM, CODE_DIM), lambda i: (i, 0)),
            pl.BlockSpec((CODEBOOK_SIZE, CODE_DIM), lambda i: (0, 0)),
        ],
        out_specs=pl.BlockSpec((TM,), lambda i: (i,)),
        out_shape=jax.ShapeDtypeStruct((N_TOKENS,), jnp.int32),
        compiler_params=pltpu.CompilerParams(
            dimension_semantics=("parallel",),
        ),
    )(x, codebook)


def _sc_gather(codebook, indices):
    info = plsc.get_sparse_core_info()
    nw = info.num_cores * info.num_subcores
    b_per_w = N_TOKENS // nw
    n_chunks = b_per_w // 128
    idx3 = indices.reshape(nw, n_chunks, 128)
    mesh = plsc.VectorSubcoreMesh(core_axis_name="c", subcore_axis_name="s")

    @functools.partial(
        pl.kernel,
        mesh=mesh,
        out_type=jax.ShapeDtypeStruct((N_TOKENS, CODE_DIM), jnp.float32),
        scratch_types=[
            pltpu.VMEM((n_chunks, 128), jnp.int32),
            pltpu.VMEM((b_per_w, CODE_DIM), jnp.float32),
            pltpu.SemaphoreType.DMA,
        ],
    )
    def gather_k(table_hbm, idx_hbm, out_hbm, idx_v, rows_v, sem):
        wid = lax.axis_index("s") * info.num_cores + lax.axis_index("c")
        pltpu.sync_copy(idx_hbm.at[wid], idx_v)
        copies = []
        for c in range(n_chunks):
            copies.append(pltpu.async_copy(
                table_hbm.at[idx_v.at[c]],
                rows_v.at[pl.ds(c * 128, 128)],
                sem,
            ))
        for cp in copies:
            cp.wait()
        pltpu.sync_copy(rows_v, out_hbm.at[pl.ds(wid * b_per_w, b_per_w)])

    return gather_k(codebook, idx3)


def kernel(x, codebook):
    codebook = jnp.asarray(codebook, dtype=jnp.float32)
    indices = _argmin_indices(x, codebook)
    quantized = _sc_gather(codebook, indices)
    return (quantized, indices)


# speedup vs baseline: 1.3657x; 1.3657x over previous
import functools

import jax
import jax.numpy as jnp
from jax import lax
from jax.experimental import pallas as pl
from jax.experimental.pallas import tpu as pltpu
from jax.experimental.pallas import tpu_sc as plsc

CODEBOOK_SIZE = 8192
CODE_DIM = 32
N_TOKENS = 16384

TM = 256
W = 4096


def _argmin_body(x_ref, cb_ref, a2_ref, b2_ref, idx_ref):
    x = x_ref[...]
    a2 = a2_ref[...]
    acc_v = None
    acc_i = None
    for w in range(CODEBOOK_SIZE // W):
        cbw = cb_ref[pl.ds(w * W, W), :]
        ab = lax.dot_general(
            x.astype(jnp.bfloat16), cbw.astype(jnp.bfloat16),
            (((1,), (1,)), ((), ())),
            preferred_element_type=jnp.float32,
        )
        b2w = b2_ref[:, pl.ds(w * W, W)]
        dist = (a2 - 2.0 * ab) + b2w
        ev = jnp.min(dist, axis=1, keepdims=True)
        iota = lax.broadcasted_iota(jnp.int32, dist.shape, 1)
        ei = jnp.min(jnp.where(dist == ev, iota, W), axis=1) + w * W
        ev = ev[:, 0]
        if acc_v is None:
            acc_v, acc_i = ev, ei
        else:
            keep = acc_v <= ev
            acc_i = jnp.where(keep, acc_i, ei)
            acc_v = jnp.where(keep, acc_v, ev)
        acc_v = acc_v.astype(jnp.bfloat16).astype(jnp.float32)
    idx_ref[...] = acc_i.astype(jnp.int32)


def _argmin_indices(x, codebook, a2, b2):
    grid = (N_TOKENS // TM,)
    return pl.pallas_call(
        _argmin_body,
        grid=grid,
        in_specs=[
            pl.BlockSpec((TM, CODE_DIM), lambda i: (i, 0)),
            pl.BlockSpec((CODEBOOK_SIZE, CODE_DIM), lambda i: (0, 0)),
            pl.BlockSpec((TM, 1), lambda i: (i, 0)),
            pl.BlockSpec((1, CODEBOOK_SIZE), lambda i: (0, 0)),
        ],
        out_specs=pl.BlockSpec((TM,), lambda i: (i,)),
        out_shape=jax.ShapeDtypeStruct((N_TOKENS,), jnp.int32),
        compiler_params=pltpu.CompilerParams(
            dimension_semantics=("parallel",),
        ),
    )(x, codebook, a2, b2)


def _sc_gather(codebook, indices):
    info = plsc.get_sparse_core_info()
    nw = info.num_cores * info.num_subcores
    b_per_w = N_TOKENS // nw
    n_chunks = b_per_w // 128
    idx3 = indices.reshape(nw, n_chunks, 128)
    mesh = plsc.VectorSubcoreMesh(core_axis_name="c", subcore_axis_name="s")

    @functools.partial(
        pl.kernel,
        mesh=mesh,
        out_type=jax.ShapeDtypeStruct((N_TOKENS, CODE_DIM), jnp.float32),
        scratch_types=[
            pltpu.VMEM((n_chunks, 128), jnp.int32),
            pltpu.VMEM((b_per_w, CODE_DIM), jnp.float32),
            pltpu.SemaphoreType.DMA,
        ],
        compiler_params=pltpu.CompilerParams(use_tc_tiling_on_sc=False),
    )
    def gather_k(table_hbm, idx_hbm, out_hbm, idx_v, rows_v, sem):
        wid = lax.axis_index("s") * info.num_cores + lax.axis_index("c")
        pltpu.sync_copy(idx_hbm.at[wid], idx_v)
        copies = []
        for c in range(n_chunks):
            copies.append(pltpu.async_copy(
                table_hbm.at[idx_v.at[c]],
                rows_v.at[pl.ds(c * 128, 128)],
                sem,
            ))
        for cp in copies:
            cp.wait()
        pltpu.sync_copy(rows_v, out_hbm.at[pl.ds(wid * b_per_w, b_per_w)])

    return gather_k(codebook, idx3)


def kernel(x, codebook):
    codebook = jnp.asarray(codebook, dtype=jnp.float32)
    a2 = jnp.sum(x ** 2, axis=1, keepdims=True)
    b2 = jnp.sum(codebook.T ** 2, axis=0, keepdims=True)
    indices = _argmin_indices(x, codebook, a2, b2)
    quantized = _sc_gather(codebook, indices)
    return (quantized, indices)


# speedup vs baseline: 1.4311x; 1.0479x over previous
import functools

import jax
import jax.numpy as jnp
from jax import lax
from jax.experimental import pallas as pl
from jax.experimental.pallas import tpu as pltpu
from jax.experimental.pallas import tpu_sc as plsc

CODEBOOK_SIZE = 8192
CODE_DIM = 32
N_TOKENS = 16384

TM = 512
W = 4096


def _argmin_body(x_ref, cb_ref, a2_ref, b2_ref, idx_ref):
    x = x_ref[...]
    a2 = a2_ref[...]
    acc_v = None
    acc_i = None
    for w in range(CODEBOOK_SIZE // W):
        cbw = cb_ref[pl.ds(w * W, W), :]
        ab = lax.dot_general(
            x.astype(jnp.bfloat16), cbw.astype(jnp.bfloat16),
            (((1,), (1,)), ((), ())),
            preferred_element_type=jnp.float32,
        )
        b2w = b2_ref[:, pl.ds(w * W, W)]
        dist = (a2 - 2.0 * ab) + b2w
        ev = jnp.min(dist, axis=1, keepdims=True)
        iota = lax.broadcasted_iota(jnp.int32, dist.shape, 1)
        ei = jnp.min(jnp.where(dist == ev, iota, W), axis=1) + w * W
        ev = ev[:, 0]
        if acc_v is None:
            acc_v, acc_i = ev, ei
        else:
            keep = acc_v <= ev
            acc_i = jnp.where(keep, acc_i, ei)
            acc_v = jnp.where(keep, acc_v, ev)
        acc_v = acc_v.astype(jnp.bfloat16).astype(jnp.float32)
    idx_ref[...] = acc_i.astype(jnp.int32)


def _argmin_indices(x, codebook, a2, b2):
    grid = (N_TOKENS // TM,)
    return pl.pallas_call(
        _argmin_body,
        grid=grid,
        in_specs=[
            pl.BlockSpec((TM, CODE_DIM), lambda i: (i, 0)),
            pl.BlockSpec((CODEBOOK_SIZE, CODE_DIM), lambda i: (0, 0)),
            pl.BlockSpec((TM, 1), lambda i: (i, 0)),
            pl.BlockSpec((1, CODEBOOK_SIZE), lambda i: (0, 0)),
        ],
        out_specs=pl.BlockSpec((TM,), lambda i: (i,)),
        out_shape=jax.ShapeDtypeStruct((N_TOKENS,), jnp.int32),
        compiler_params=pltpu.CompilerParams(
            dimension_semantics=("parallel",),
        ),
    )(x, codebook, a2, b2)


def _sc_gather(codebook, indices):
    info = plsc.get_sparse_core_info()
    nw = info.num_cores * info.num_subcores
    b_per_w = N_TOKENS // nw
    n_chunks = b_per_w // 128
    idx3 = indices.reshape(nw, n_chunks, 128)
    mesh = plsc.VectorSubcoreMesh(core_axis_name="c", subcore_axis_name="s")

    @functools.partial(
        pl.kernel,
        mesh=mesh,
        out_type=jax.ShapeDtypeStruct((N_TOKENS, CODE_DIM), jnp.float32),
        scratch_types=[
            pltpu.VMEM((n_chunks, 128), jnp.int32),
            pltpu.VMEM((b_per_w, CODE_DIM), jnp.float32),
            pltpu.SemaphoreType.DMA,
        ],
        compiler_params=pltpu.CompilerParams(use_tc_tiling_on_sc=False),
    )
    def gather_k(table_hbm, idx_hbm, out_hbm, idx_v, rows_v, sem):
        wid = lax.axis_index("s") * info.num_cores + lax.axis_index("c")
        pltpu.sync_copy(idx_hbm.at[wid], idx_v)
        copies = []
        for c in range(n_chunks):
            copies.append(pltpu.async_copy(
                table_hbm.at[idx_v.at[c]],
                rows_v.at[pl.ds(c * 128, 128)],
                sem,
            ))
        for cp in copies:
            cp.wait()
        pltpu.sync_copy(rows_v, out_hbm.at[pl.ds(wid * b_per_w, b_per_w)])

    return gather_k(codebook, idx3)


def kernel(x, codebook):
    codebook = jnp.asarray(codebook, dtype=jnp.float32)
    a2 = jnp.sum(x ** 2, axis=1, keepdims=True)
    b2 = jnp.sum(codebook.T ** 2, axis=0, keepdims=True)
    indices = _argmin_indices(x, codebook, a2, b2)
    quantized = _sc_gather(codebook, indices)
    return (quantized, indices)


# speedup vs baseline: 1.4560x; 1.0174x over previous
import functools

import jax
import jax.numpy as jnp
from jax import lax
from jax.experimental import pallas as pl
from jax.experimental.pallas import tpu as pltpu
from jax.experimental.pallas import tpu_sc as plsc

CODEBOOK_SIZE = 8192
CODE_DIM = 32
N_TOKENS = 16384

TM = 1024
W = 4096


def _argmin_body(x_ref, cb_ref, a2_ref, b2_ref, idx_ref):
    x = x_ref[...]
    a2 = a2_ref[...]
    acc_v = None
    acc_i = None
    for w in range(CODEBOOK_SIZE // W):
        cbw = cb_ref[pl.ds(w * W, W), :]
        ab = lax.dot_general(
            x.astype(jnp.bfloat16), cbw.astype(jnp.bfloat16),
            (((1,), (1,)), ((), ())),
            preferred_element_type=jnp.float32,
        )
        b2w = b2_ref[:, pl.ds(w * W, W)]
        dist = (a2 - 2.0 * ab) + b2w
        ev = jnp.min(dist, axis=1, keepdims=True)
        iota = lax.broadcasted_iota(jnp.int32, dist.shape, 1)
        ei = jnp.min(jnp.where(dist == ev, iota, W), axis=1) + w * W
        ev = ev[:, 0]
        if acc_v is None:
            acc_v, acc_i = ev, ei
        else:
            keep = acc_v <= ev
            acc_i = jnp.where(keep, acc_i, ei)
            acc_v = jnp.where(keep, acc_v, ev)
        acc_v = acc_v.astype(jnp.bfloat16).astype(jnp.float32)
    idx_ref[...] = acc_i.astype(jnp.int32)


def _argmin_indices(x, codebook, a2, b2):
    grid = (N_TOKENS // TM,)
    return pl.pallas_call(
        _argmin_body,
        grid=grid,
        in_specs=[
            pl.BlockSpec((TM, CODE_DIM), lambda i: (i, 0)),
            pl.BlockSpec((CODEBOOK_SIZE, CODE_DIM), lambda i: (0, 0)),
            pl.BlockSpec((TM, 1), lambda i: (i, 0)),
            pl.BlockSpec((1, CODEBOOK_SIZE), lambda i: (0, 0)),
        ],
        out_specs=pl.BlockSpec((TM,), lambda i: (i,)),
        out_shape=jax.ShapeDtypeStruct((N_TOKENS,), jnp.int32),
        compiler_params=pltpu.CompilerParams(
            dimension_semantics=("parallel",),
        ),
    )(x, codebook, a2, b2)


def _sc_gather(codebook, indices):
    info = plsc.get_sparse_core_info()
    nw = info.num_cores * info.num_subcores
    b_per_w = N_TOKENS // nw
    n_chunks = b_per_w // 128
    idx3 = indices.reshape(nw, n_chunks, 128)
    mesh = plsc.VectorSubcoreMesh(core_axis_name="c", subcore_axis_name="s")

    @functools.partial(
        pl.kernel,
        mesh=mesh,
        out_type=jax.ShapeDtypeStruct((N_TOKENS, CODE_DIM), jnp.float32),
        scratch_types=[
            pltpu.VMEM((n_chunks, 128), jnp.int32),
            pltpu.VMEM((b_per_w, CODE_DIM), jnp.float32),
            pltpu.SemaphoreType.DMA,
        ],
        compiler_params=pltpu.CompilerParams(use_tc_tiling_on_sc=False),
    )
    def gather_k(table_hbm, idx_hbm, out_hbm, idx_v, rows_v, sem):
        wid = lax.axis_index("s") * info.num_cores + lax.axis_index("c")
        pltpu.sync_copy(idx_hbm.at[wid], idx_v)
        copies = []
        for c in range(n_chunks):
            copies.append(pltpu.async_copy(
                table_hbm.at[idx_v.at[c]],
                rows_v.at[pl.ds(c * 128, 128)],
                sem,
            ))
        for cp in copies:
            cp.wait()
        pltpu.sync_copy(rows_v, out_hbm.at[pl.ds(wid * b_per_w, b_per_w)])

    return gather_k(codebook, idx3)


def kernel(x, codebook):
    codebook = jnp.asarray(codebook, dtype=jnp.float32)
    a2 = jnp.sum(x ** 2, axis=1, keepdims=True)
    b2 = jnp.sum(codebook.T ** 2, axis=0, keepdims=True)
    indices = _argmin_indices(x, codebook, a2, b2)
    quantized = _sc_gather(codebook, indices)
    return (quantized, indices)


# speedup vs baseline: 1.4915x; 1.0244x over previous
import functools

import jax
import jax.numpy as jnp
from jax import lax
from jax.experimental import pallas as pl
from jax.experimental.pallas import tpu as pltpu
from jax.experimental.pallas import tpu_sc as plsc

CODEBOOK_SIZE = 8192
CODE_DIM = 32
N_TOKENS = 16384

TM = 2048
W = 4096


def _argmin_body(x_ref, cb_ref, a2_ref, b2_ref, idx_ref):
    x = x_ref[...]
    a2 = a2_ref[...]
    acc_v = None
    acc_i = None
    for w in range(CODEBOOK_SIZE // W):
        cbw = cb_ref[pl.ds(w * W, W), :]
        ab = lax.dot_general(
            x.astype(jnp.bfloat16), cbw.astype(jnp.bfloat16),
            (((1,), (1,)), ((), ())),
            preferred_element_type=jnp.float32,
        )
        b2w = b2_ref[:, pl.ds(w * W, W)]
        dist = (a2 - 2.0 * ab) + b2w
        ev = jnp.min(dist, axis=1, keepdims=True)
        iota = lax.broadcasted_iota(jnp.int32, dist.shape, 1)
        ei = jnp.min(jnp.where(dist == ev, iota, W), axis=1) + w * W
        ev = ev[:, 0]
        if acc_v is None:
            acc_v, acc_i = ev, ei
        else:
            keep = acc_v <= ev
            acc_i = jnp.where(keep, acc_i, ei)
            acc_v = jnp.where(keep, acc_v, ev)
        acc_v = acc_v.astype(jnp.bfloat16).astype(jnp.float32)
    idx_ref[...] = acc_i.astype(jnp.int32)


def _argmin_indices(x, codebook, a2, b2):
    grid = (N_TOKENS // TM,)
    return pl.pallas_call(
        _argmin_body,
        grid=grid,
        in_specs=[
            pl.BlockSpec((TM, CODE_DIM), lambda i: (i, 0)),
            pl.BlockSpec((CODEBOOK_SIZE, CODE_DIM), lambda i: (0, 0)),
            pl.BlockSpec((TM, 1), lambda i: (i, 0)),
            pl.BlockSpec((1, CODEBOOK_SIZE), lambda i: (0, 0)),
        ],
        out_specs=pl.BlockSpec((TM,), lambda i: (i,)),
        out_shape=jax.ShapeDtypeStruct((N_TOKENS,), jnp.int32),
        compiler_params=pltpu.CompilerParams(
            dimension_semantics=("parallel",),
        ),
    )(x, codebook, a2, b2)


def _sc_gather(codebook, indices):
    info = plsc.get_sparse_core_info()
    nw = info.num_cores * info.num_subcores
    b_per_w = N_TOKENS // nw
    n_chunks = b_per_w // 128
    idx3 = indices.reshape(nw, n_chunks, 128)
    mesh = plsc.VectorSubcoreMesh(core_axis_name="c", subcore_axis_name="s")

    @functools.partial(
        pl.kernel,
        mesh=mesh,
        out_type=jax.ShapeDtypeStruct((N_TOKENS, CODE_DIM), jnp.float32),
        scratch_types=[
            pltpu.VMEM((n_chunks, 128), jnp.int32),
            pltpu.VMEM((b_per_w, CODE_DIM), jnp.float32),
            pltpu.SemaphoreType.DMA,
        ],
        compiler_params=pltpu.CompilerParams(use_tc_tiling_on_sc=False),
    )
    def gather_k(table_hbm, idx_hbm, out_hbm, idx_v, rows_v, sem):
        wid = lax.axis_index("s") * info.num_cores + lax.axis_index("c")
        pltpu.sync_copy(idx_hbm.at[wid], idx_v)
        copies = []
        for c in range(n_chunks):
            copies.append(pltpu.async_copy(
                table_hbm.at[idx_v.at[c]],
                rows_v.at[pl.ds(c * 128, 128)],
                sem,
            ))
        for cp in copies:
            cp.wait()
        pltpu.sync_copy(rows_v, out_hbm.at[pl.ds(wid * b_per_w, b_per_w)])

    return gather_k(codebook, idx3)


def kernel(x, codebook):
    codebook = jnp.asarray(codebook, dtype=jnp.float32)
    a2 = jnp.sum(x ** 2, axis=1, keepdims=True)
    b2 = jnp.sum(codebook.T ** 2, axis=0, keepdims=True)
    indices = _argmin_indices(x, codebook, a2, b2)
    quantized = _sc_gather(codebook, indices)
    return (quantized, indices)
